# Initial kernel scaffold; baseline (speedup 1.0000x reference)
import jax
import jax.numpy as jnp
from jax.experimental import pallas as pl


def kernel(idx, feat_bias, feat_vect):
    raise NotImplementedError("write your pallas kernel here")


# speedup vs baseline: 1.3661x; 1.3661x over previous
import jax
import jax.numpy as jnp
from jax import lax
from jax.experimental import pallas as pl
from jax.experimental.pallas import tpu as pltpu
from jax.experimental.pallas import tpu_sc as plsc

N_FEATURES = 1_000_000
N_DIM = 16
BATCH = 16384
N_FIELDS = 26

NC = 2
NS = 16
NW = NC * NS
ROWS_PER_W = BATCH // NW
CHUNK = 64
N_CHUNKS = ROWS_PER_W // CHUNK
IDX_PER_CHUNK = CHUNK * N_FIELDS
IDX_TILE = 128
N_TILES = IDX_PER_CHUNK // IDX_TILE
GROUPS = CHUNK // 16


def _shuf(x, perm):
    return lax.gather(
        x, perm[:, None],
        lax.GatherDimensionNumbers(
            offset_dims=(), collapsed_slice_dims=(0,), start_index_map=(0,)),
        slice_sizes=(1,),
        mode=lax.GatherScatterMode.PROMISE_IN_BOUNDS)


def _fm_body(idx_hbm, bias_hbm, vect_hbm, out_hbm,
             idx_v, rows_v, bias_v, out_v, sem_rows, sem_bias):
    wid = lax.axis_index("s") * NC + lax.axis_index("c")
    lanes = lax.iota(jnp.int32, 16)
    tiles_per_w = ROWS_PER_W * N_FIELDS // IDX_TILE

    pltpu.sync_copy(idx_hbm.at[pl.ds(wid * tiles_per_w, tiles_per_w)], idx_v)

    def chunk_body(c, carry):
        descs = []
        for i in range(N_TILES):
            descs.append(pltpu.async_copy(
                vect_hbm.at[idx_v.at[c * N_TILES + i]],
                rows_v.at[pl.ds(i * IDX_TILE, IDX_TILE)],
                sem_rows))
            descs.append(pltpu.async_copy(
                bias_hbm.at[idx_v.at[c * N_TILES + i]],
                bias_v.at[pl.ds(i * IDX_TILE, IDX_TILE)],
                sem_bias))
        for d in descs:
            d.wait()

        def group_body(g, carry2):
            ts = []
            for j in range(16):
                base = (g * 16 + j) * N_FIELDS
                s = jnp.zeros((16,), jnp.float32)
                q = jnp.zeros((16,), jnp.float32)
                for f in range(N_FIELDS):
                    v = rows_v[base + f]
                    s = s + v
                    q = q + v * v
                b0 = bias_v[pl.ds(base, 16)]
                b1 = bias_v[pl.ds(base + 16, 16)]
                b1 = jnp.where(lanes < N_FIELDS - 16, b1, 0.0)
                ts.append(0.5 * (s * s - q) + b0 + b1)
            for k in range(4):
                step = 1 << k
                perm = lanes ^ step
                keep = ((lanes >> k) & 1) == 0
                nxt = []
                for p in range(0, len(ts), 2):
                    a = ts[p] + _shuf(ts[p], perm)
                    b = ts[p + 1] + _shuf(ts[p + 1], perm)
                    nxt.append(jnp.where(keep, a, b))
                ts = nxt
            out_v[pl.ds(g * 16, 16)] = ts[0]
            return carry2

        lax.fori_loop(0, GROUPS, group_body, 0)
        pltpu.sync_copy(out_v,
                        out_hbm.at[pl.ds(wid * ROWS_PER_W + c * CHUNK, CHUNK)])
        return carry

    lax.fori_loop(0, N_CHUNKS, chunk_body, 0)


def kernel(idx, feat_bias, feat_vect):
    idx2 = idx.astype(jnp.int32).reshape(BATCH * N_FIELDS // IDX_TILE, IDX_TILE)
    bias_flat = feat_bias.reshape(N_FEATURES)
    mesh = plsc.VectorSubcoreMesh(core_axis_name="c", subcore_axis_name="s")
    k = pl.kernel(
        _fm_body,
        mesh=mesh,
        compiler_params=pltpu.CompilerParams(use_tc_tiling_on_sc=False),
        out_type=jax.ShapeDtypeStruct((BATCH,), jnp.float32),
        scratch_types=[
            pltpu.VMEM((ROWS_PER_W * N_FIELDS // IDX_TILE, IDX_TILE), jnp.int32),
            pltpu.VMEM((IDX_PER_CHUNK, N_DIM), jnp.float32),
            pltpu.VMEM((IDX_PER_CHUNK + 32,), jnp.float32),
            pltpu.VMEM((CHUNK,), jnp.float32),
            pltpu.SemaphoreType.DMA,
            pltpu.SemaphoreType.DMA,
        ],
    )
    return k(idx2, bias_flat, feat_vect)
